# baseline (speedup 1.0000x reference)
import functools

import jax
import jax.numpy as jnp
from jax import lax
from jax.experimental import pallas as pl
from jax.experimental.pallas import tpu as pltpu
from jax.experimental.pallas import tpu_sc as plsc

_B = 4096
_L = 200
_D = 64


def _make_sc_gather():
    info = plsc.get_sparse_core_info()
    nw = info.num_cores * info.num_subcores
    b_per_w = _B // nw
    mesh = plsc.VectorSubcoreMesh(core_axis_name="c", subcore_axis_name="s")

    @functools.partial(
        pl.kernel,
        mesh=mesh,
        out_type=jax.ShapeDtypeStruct((_B, _D), jnp.float32),
        scratch_types=[
            pltpu.VMEM((b_per_w,), jnp.int32),
            pltpu.VMEM((b_per_w, _D), jnp.float32),
            pltpu.SemaphoreType.DMA,
        ],
        compiler_params=pltpu.CompilerParams(needs_layout_passes=False),
    )
    def sc_gather(table_hbm, idx_hbm, out_hbm, idx_v, rows_v, sem):
        wid = lax.axis_index("s") * info.num_cores + lax.axis_index("c")
        base = wid * b_per_w
        pltpu.sync_copy(idx_hbm.at[pl.ds(base, b_per_w)], idx_v)
        iota16 = lax.iota(jnp.int32, 16)
        copies = []
        for j in range(b_per_w):
            chunk = idx_v[pl.ds((j // 16) * 16, 16)]
            row_s = jnp.sum(jnp.where(iota16 == (j % 16), chunk, 0), axis=0)
            c = pltpu.make_async_copy(
                table_hbm.at[pl.ds(row_s, 1)],
                rows_v.at[pl.ds(j, 1)],
                sem,
            )
            c.start()
            copies.append(c)
        for c in copies:
            c.wait()
        pltpu.sync_copy(rows_v, out_hbm.at[pl.ds(base, b_per_w)])

    return sc_gather


_sc_gather = _make_sc_gather()

_BB = 128


def _copy_body(feat_ref, idx_in_ref, out_ref, idx_out_ref, sem):
    @pl.when(pl.program_id(0) == 0)
    def _():
        c = pltpu.make_async_copy(idx_in_ref, idx_out_ref, sem)
        c.start()
        c.wait()

    out_ref[1:, :, :] = feat_ref[...]


_copy_feat = pl.pallas_call(
    _copy_body,
    grid=(_B // _BB,),
    in_specs=[
        pl.BlockSpec((_L, _D, _BB), lambda i: (0, 0, i)),
        pl.BlockSpec(memory_space=pl.ANY),
    ],
    out_specs=[
        pl.BlockSpec((_L + 1, _D, _BB), lambda i: (0, 0, i)),
        pl.BlockSpec(memory_space=pl.ANY),
    ],
    out_shape=[
        jax.ShapeDtypeStruct((_L + 1, _D, _B), jnp.float32),
        jax.ShapeDtypeStruct((_B,), jnp.int32),
    ],
    scratch_shapes=[pltpu.SemaphoreType.DMA],
)


def _patch_body(emb_ref, prev_ref, out_ref):
    del prev_ref
    out_ref[...] = emb_ref[...]


_patch = pl.pallas_call(
    _patch_body,
    grid=(1,),
    in_specs=[
        pl.BlockSpec((1, _D, _B), lambda i: (0, 0, 0)),
        pl.BlockSpec(memory_space=pl.ANY),
    ],
    out_specs=pl.BlockSpec((1, _D, _B), lambda i: (0, 0, 0)),
    out_shape=jax.ShapeDtypeStruct((_L + 1, _D, _B), jnp.float32),
    input_output_aliases={1: 0},
)


def kernel(feature, index, table):
    idx = index.reshape(-1).astype(jnp.int32)
    feat_t = jnp.transpose(feature, (1, 2, 0))
    out_t, idx_pass = _copy_feat(feat_t, idx)
    emb = _sc_gather(table, idx_pass)
    emb_t = jnp.transpose(emb)[None]
    out_t = _patch(emb_t, out_t)
    return jnp.transpose(out_t, (2, 0, 1))


# speedup vs baseline: 2.4774x; 2.4774x over previous
import functools

import jax
import jax.numpy as jnp
from jax import lax
from jax.experimental import pallas as pl
from jax.experimental.pallas import tpu as pltpu
from jax.experimental.pallas import tpu_sc as plsc

_B = 4096
_L = 200
_D = 64


def _make_sc_gather():
    info = plsc.get_sparse_core_info()
    nw = info.num_cores * info.num_subcores
    b_per_w = _B // nw
    mesh = plsc.VectorSubcoreMesh(core_axis_name="c", subcore_axis_name="s")

    ring = 8

    @functools.partial(
        pl.kernel,
        mesh=mesh,
        out_type=jax.ShapeDtypeStruct((_D, _B), jnp.float32),
        scratch_types=[
            pltpu.VMEM((b_per_w,), jnp.int32),
            pltpu.VMEM((ring, _D, 128), jnp.float32),
            pltpu.VMEM((_D, b_per_w), jnp.float32),
            pltpu.SemaphoreType.DMA,
        ],
        compiler_params=pltpu.CompilerParams(needs_layout_passes=False),
    )
    def sc_gather(tablet_hbm, idx_hbm, out_hbm, idx_v, ring_v, embt_v, sem):
        wid = lax.axis_index("s") * info.num_cores + lax.axis_index("c")
        base = wid * b_per_w
        pltpu.sync_copy(idx_hbm.at[pl.ds(base, b_per_w)], idx_v)
        iota16 = lax.iota(jnp.int32, 16)

        def col_of(j):
            chunk = idx_v[pl.ds((j // 16) * 16, 16)]
            return jnp.sum(jnp.where(iota16 == (j % 16), chunk, 0), axis=0)

        def issue(j):
            col_s = col_of(j)
            tile_s = pl.multiple_of((col_s >> 7) << 7, 128)
            c = pltpu.make_async_copy(
                tablet_hbm.at[:, pl.ds(tile_s, 128)],
                ring_v.at[j % ring],
                sem,
            )
            c.start()
            return c

        copies = [issue(j) for j in range(ring)]
        for j in range(b_per_w):
            copies[j % ring].wait()
            lane_s = col_of(j) & 127
            lane_vec = jnp.full((16,), lane_s, jnp.int32)
            col_vec = jnp.full((16,), j, jnp.int32)
            for g in range(_D // 16):
                d16 = iota16 + g * 16
                vals = plsc.load_gather(ring_v.at[j % ring], [d16, lane_vec])
                plsc.store_scatter(embt_v, [d16, col_vec], vals)
            if j + ring < b_per_w:
                copies[(j + ring) % ring] = issue(j + ring)
        pltpu.sync_copy(embt_v, out_hbm.at[:, pl.ds(base, b_per_w)])

    return sc_gather


_sc_gather = _make_sc_gather()

_BB = 128


def _copy_body(feat_ref, idx_in_ref, out_ref, idx_out_ref, sem):
    @pl.when(pl.program_id(0) == 0)
    def _():
        c = pltpu.make_async_copy(idx_in_ref, idx_out_ref, sem)
        c.start()
        c.wait()

    out_ref[1:, :, :] = feat_ref[...]


_copy_feat = pl.pallas_call(
    _copy_body,
    grid=(_B // _BB,),
    in_specs=[
        pl.BlockSpec((_L, _D, _BB), lambda i: (0, 0, i)),
        pl.BlockSpec(memory_space=pl.ANY),
    ],
    out_specs=[
        pl.BlockSpec((_L + 1, _D, _BB), lambda i: (0, 0, i)),
        pl.BlockSpec(memory_space=pl.ANY),
    ],
    out_shape=[
        jax.ShapeDtypeStruct((_L + 1, _D, _B), jnp.float32),
        jax.ShapeDtypeStruct((_B,), jnp.int32),
    ],
    scratch_shapes=[pltpu.SemaphoreType.DMA],
)


def _patch_body(emb_ref, prev_ref, out_ref):
    del prev_ref
    out_ref[...] = emb_ref[...]


_patch = pl.pallas_call(
    _patch_body,
    grid=(1,),
    in_specs=[
        pl.BlockSpec((1, _D, _B), lambda i: (0, 0, 0)),
        pl.BlockSpec(memory_space=pl.ANY),
    ],
    out_specs=pl.BlockSpec((1, _D, _B), lambda i: (0, 0, 0)),
    out_shape=jax.ShapeDtypeStruct((_L + 1, _D, _B), jnp.float32),
    input_output_aliases={1: 0},
)


def kernel(feature, index, table):
    idx = index.reshape(-1).astype(jnp.int32)
    feat_t = jnp.transpose(feature, (1, 2, 0))
    table_t = jnp.transpose(table)
    out_t, idx_pass = _copy_feat(feat_t, idx)
    emb_t = _sc_gather(table_t, idx_pass)[None]
    out_t = _patch(emb_t, out_t)
    return jnp.transpose(out_t, (2, 0, 1))


# speedup vs baseline: 2.5632x; 1.0346x over previous
import functools

import jax
import jax.numpy as jnp
from jax import lax
from jax.experimental import pallas as pl
from jax.experimental.pallas import tpu as pltpu
from jax.experimental.pallas import tpu_sc as plsc

_B = 4096
_L = 200
_D = 64


def _make_sc_gather():
    info = plsc.get_sparse_core_info()
    nw = info.num_cores * info.num_subcores
    b_per_w = _B // nw
    mesh = plsc.VectorSubcoreMesh(core_axis_name="c", subcore_axis_name="s")

    ring = 8

    @functools.partial(
        pl.kernel,
        mesh=mesh,
        out_type=jax.ShapeDtypeStruct((_D, _B), jnp.float32),
        scratch_types=[
            pltpu.VMEM((b_per_w,), jnp.int32),
            pltpu.VMEM((ring, _D, 128), jnp.float32),
            pltpu.VMEM((_D, b_per_w), jnp.float32),
            pltpu.SemaphoreType.DMA,
        ],
        compiler_params=pltpu.CompilerParams(needs_layout_passes=False),
    )
    def sc_gather(tablet_hbm, idx_hbm, out_hbm, idx_v, ring_v, embt_v, sem):
        wid = lax.axis_index("s") * info.num_cores + lax.axis_index("c")
        base = wid * b_per_w
        pltpu.sync_copy(idx_hbm.at[pl.ds(base, b_per_w)], idx_v)
        iota16 = lax.iota(jnp.int32, 16)

        def col_of(j):
            chunk = idx_v[pl.ds((j // 16) * 16, 16)]
            return jnp.sum(jnp.where(iota16 == (j % 16), chunk, 0), axis=0)

        def issue(j):
            col_s = col_of(j)
            tile_s = pl.multiple_of((col_s >> 7) << 7, 128)
            c = pltpu.make_async_copy(
                tablet_hbm.at[:, pl.ds(tile_s, 128)],
                ring_v.at[j % ring],
                sem,
            )
            c.start()
            return c

        copies = [issue(j) for j in range(ring)]
        for j in range(b_per_w):
            copies[j % ring].wait()
            lane_s = col_of(j) & 127
            lane_vec = jnp.full((16,), lane_s, jnp.int32)
            col_vec = jnp.full((16,), j, jnp.int32)
            for g in range(_D // 16):
                d16 = iota16 + g * 16
                vals = plsc.load_gather(ring_v.at[j % ring], [d16, lane_vec])
                plsc.store_scatter(embt_v, [d16, col_vec], vals)
            if j + ring < b_per_w:
                copies[(j + ring) % ring] = issue(j + ring)
        pltpu.sync_copy(embt_v, out_hbm.at[:, pl.ds(base, b_per_w)])

    return sc_gather


_sc_gather = _make_sc_gather()

_BB = 128


def _copy_body(feat_ref, out_ref):
    out_ref[1:, :, :] = feat_ref[...]


_copy_feat = pl.pallas_call(
    _copy_body,
    grid=(_B // _BB,),
    in_specs=[pl.BlockSpec((_L, _D, _BB), lambda i: (0, 0, i))],
    out_specs=pl.BlockSpec((_L + 1, _D, _BB), lambda i: (0, 0, i)),
    out_shape=jax.ShapeDtypeStruct((_L + 1, _D, _B), jnp.float32),
)


def _patch_body(emb_ref, prev_ref, out_ref):
    del prev_ref
    out_ref[...] = emb_ref[...]


_patch = pl.pallas_call(
    _patch_body,
    grid=(1,),
    in_specs=[
        pl.BlockSpec((1, _D, _B), lambda i: (0, 0, 0)),
        pl.BlockSpec(memory_space=pl.ANY),
    ],
    out_specs=pl.BlockSpec((1, _D, _B), lambda i: (0, 0, 0)),
    out_shape=jax.ShapeDtypeStruct((_L + 1, _D, _B), jnp.float32),
    input_output_aliases={1: 0},
)


def kernel(feature, index, table):
    idx = index.reshape(-1).astype(jnp.int32)
    feat_t = jnp.transpose(feature, (1, 2, 0))
    table_t = jnp.transpose(table)
    emb_t = _sc_gather(table_t, idx)[None]
    out_t = _copy_feat(feat_t)
    out_t = _patch(emb_t, out_t)
    return jnp.transpose(out_t, (2, 0, 1))


# speedup vs baseline: 2.5660x; 1.0011x over previous
import functools

import jax
import jax.numpy as jnp
from jax import lax
from jax.experimental import pallas as pl
from jax.experimental.pallas import tpu as pltpu
from jax.experimental.pallas import tpu_sc as plsc

_B = 4096
_L = 200
_D = 64


def _make_sc_gather():
    info = plsc.get_sparse_core_info()
    nw = info.num_cores * info.num_subcores
    b_per_w = _B // nw
    mesh = plsc.VectorSubcoreMesh(core_axis_name="c", subcore_axis_name="s")

    ring = 8

    @functools.partial(
        pl.kernel,
        mesh=mesh,
        out_type=jax.ShapeDtypeStruct((_D, _B), jnp.float32),
        scratch_types=[
            pltpu.VMEM((b_per_w,), jnp.int32),
            pltpu.VMEM((ring, _D, 128), jnp.float32),
            pltpu.VMEM((_D, b_per_w), jnp.float32),
            pltpu.SemaphoreType.DMA,
        ],
        compiler_params=pltpu.CompilerParams(
            needs_layout_passes=False, disable_bounds_checks=True
        ),
    )
    def sc_gather(tablet_hbm, idx_hbm, out_hbm, idx_v, ring_v, embt_v, sem):
        wid = lax.axis_index("s") * info.num_cores + lax.axis_index("c")
        base = wid * b_per_w
        pltpu.sync_copy(idx_hbm.at[pl.ds(base, b_per_w)], idx_v)
        iota16 = lax.iota(jnp.int32, 16)

        def col_of(j):
            chunk = idx_v[pl.ds((j // 16) * 16, 16)]
            return jnp.sum(jnp.where(iota16 == (j % 16), chunk, 0), axis=0)

        def issue(j):
            col_s = col_of(j)
            tile_s = pl.multiple_of((col_s >> 7) << 7, 128)
            c = pltpu.make_async_copy(
                tablet_hbm.at[:, pl.ds(tile_s, 128)],
                ring_v.at[j % ring],
                sem,
            )
            c.start()
            return c

        copies = [issue(j) for j in range(ring)]
        for j in range(b_per_w):
            copies[j % ring].wait()
            lane_s = col_of(j) & 127
            lane_vec = jnp.full((16,), lane_s, jnp.int32)
            col_vec = jnp.full((16,), j, jnp.int32)
            for g in range(_D // 16):
                d16 = iota16 + g * 16
                vals = plsc.load_gather(ring_v.at[j % ring], [d16, lane_vec])
                plsc.store_scatter(embt_v, [d16, col_vec], vals)
            if j + ring < b_per_w:
                copies[(j + ring) % ring] = issue(j + ring)
        pltpu.sync_copy(embt_v, out_hbm.at[:, pl.ds(base, b_per_w)])

    return sc_gather


_sc_gather = _make_sc_gather()

_BB = 128


def _copy_body(feat_ref, out_ref):
    out_ref[1:, :, :] = feat_ref[...]


_copy_feat = pl.pallas_call(
    _copy_body,
    grid=(_B // _BB,),
    in_specs=[pl.BlockSpec((_L, _D, _BB), lambda i: (0, 0, i))],
    out_specs=pl.BlockSpec((_L + 1, _D, _BB), lambda i: (0, 0, i)),
    out_shape=jax.ShapeDtypeStruct((_L + 1, _D, _B), jnp.float32),
)


def _patch_body(emb_ref, prev_ref, out_ref):
    del prev_ref
    out_ref[...] = emb_ref[...]


_patch = pl.pallas_call(
    _patch_body,
    grid=(1,),
    in_specs=[
        pl.BlockSpec((1, _D, _B), lambda i: (0, 0, 0)),
        pl.BlockSpec(memory_space=pl.ANY),
    ],
    out_specs=pl.BlockSpec((1, _D, _B), lambda i: (0, 0, 0)),
    out_shape=jax.ShapeDtypeStruct((_L + 1, _D, _B), jnp.float32),
    input_output_aliases={1: 0},
)


def kernel(feature, index, table):
    idx = index.reshape(-1).astype(jnp.int32)
    feat_t = jnp.transpose(feature, (1, 2, 0))
    table_t = jnp.transpose(table)
    emb_t = _sc_gather(table_t, idx)[None]
    out_t = _copy_feat(feat_t)
    out_t = _patch(emb_t, out_t)
    return jnp.transpose(out_t, (2, 0, 1))
